# baseline (speedup 1.0000x reference)
import functools
import jax
import jax.numpy as jnp
from jax import lax
from jax.experimental import pallas as pl
from jax.experimental.pallas import tpu as pltpu
from jax.experimental.pallas import tpu_sc as plsc

_LN2 = 0.6931471805599453
_CUTOFF = 5.0

_NW = 32
_R = 2
_LANES = 16


def _ssp(v):
    return jax.nn.softplus(v) - _LN2


def _proj_body(x_ref, Win_ref, y_ref):
    y_ref[...] = jnp.dot(x_ref[...], Win_ref[...],
                         preferred_element_type=jnp.float32)


def _filter_body(f_ref, r_ref, mask_ref, W1_ref, b1_ref, W2_ref, b2_ref,
                 wf_ref):
    Gb, K, Fs = f_ref.shape
    f = f_ref[...].reshape(Gb * K, Fs).astype(jnp.bfloat16)
    h = _ssp(jnp.dot(f, W1_ref[...], preferred_element_type=jnp.float32)
             + b1_ref[...])
    wf = jnp.dot(h.astype(jnp.bfloat16), W2_ref[...],
                 preferred_element_type=jnp.float32) + b2_ref[...]
    m = jnp.where(r_ref[...] <= _CUTOFF, 1.0, 0.0) * mask_ref[...]
    wf_ref[...] = wf.reshape(Gb, K, -1) * m[:, :, None]


def _sc_body(y_hbm, wf_hbm, nb_hbm, out_hbm,
             idx_v, yg_v, wf_v, out_v, sem0, sem1, *, N, K, Ff, RPW):
    wid = lax.axis_index("c") * 16 + lax.axis_index("s")
    base_row = wid * RPW
    base_edge = base_row * K
    RK = _R * K
    G = RPW // _R
    NC = Ff // _LANES

    pltpu.sync_copy(nb_hbm.at[pl.ds(base_edge, RPW * K)], idx_v)
    bvec = jnp.full((_LANES,), (base_row // N) * N, jnp.int32)

    @pl.loop(0, (RPW * K) // _LANES)
    def _(j):
        sl = pl.ds(j * _LANES, _LANES)
        idx_v[sl] = idx_v[sl] + bvec

    sems = (sem0, sem1)

    def fire(g, slot):
        pltpu.async_copy(wf_hbm.at[pl.ds((base_edge + g * RK) * Ff, RK * Ff)],
                         wf_v.at[slot], sems[slot])
        pltpu.async_copy(y_hbm.at[idx_v.at[pl.ds(g * RK, RK)]],
                         yg_v.at[slot], sems[slot])

    def drain(slot):
        pltpu.make_async_copy(wf_hbm.at[pl.ds(0, RK * Ff)], wf_v.at[slot],
                              sems[slot]).wait()
        pltpu.make_async_copy(y_hbm.at[pl.ds(0, RK)], yg_v.at[slot],
                              sems[slot]).wait()

    def compute(g, slot):
        for r in range(_R):
            def kstep(k2, accs):
                out = list(accs)
                for dk in range(2):
                    e = r * K + k2 * 2 + dk
                    for c in range(NC):
                        w = wf_v[slot, pl.ds(e * Ff + c * _LANES, _LANES)]
                        yv = yg_v[slot, e, pl.ds(c * _LANES, _LANES)]
                        out[c] = out[c] + yv * w
                return tuple(out)
            accs = lax.fori_loop(
                0, K // 2, kstep,
                tuple(jnp.zeros((_LANES,), jnp.float32) for _ in range(NC)))
            row = g * _R + r
            for c in range(NC):
                out_v[pl.ds(row * Ff + c * _LANES, _LANES)] = accs[c]

    fire(0, 0)

    @pl.loop(0, G, step=2)
    def _(g0):
        for b in range(2):
            g = g0 + b

            @pl.when(g + 1 < G)
            def _():
                fire(g + 1, 1 - b)

            drain(b)
            compute(g, b)

    pltpu.sync_copy(out_v, out_hbm.at[pl.ds(base_row * Ff, RPW * Ff)])


def _post_body(a_ref, Wout_ref, bout_ref, Wd_ref, bd_ref, o_ref):
    z = _ssp(jnp.dot(a_ref[...], Wout_ref[...],
                     preferred_element_type=jnp.float32) + bout_ref[...])
    o_ref[...] = jnp.dot(z, Wd_ref[...],
                         preferred_element_type=jnp.float32) + bd_ref[...]


@jax.jit
def kernel(x, r_ij, neighbors, neighbor_mask, f_ij,
           W1, b1, W2, b2, Win, Wout, bout, Wd, bd):
    B, N, K = neighbors.shape
    Din = x.shape[-1]
    Fs = f_ij.shape[-1]
    Ff = W2.shape[-1]
    Dout = Wd.shape[-1]
    NE = B * N * K
    RPW = (B * N) // _NW

    y = pl.pallas_call(
        _proj_body,
        in_specs=[pl.BlockSpec((B * N, Din), lambda: (0, 0)),
                  pl.BlockSpec((Din, Ff), lambda: (0, 0))],
        out_specs=pl.BlockSpec((B * N, Ff), lambda: (0, 0)),
        out_shape=jax.ShapeDtypeStruct((B * N, Ff), jnp.float32),
    )(x.reshape(B * N, Din), Win)

    Gb = 64
    T = (B * N) // Gb
    full = lambda s: pl.BlockSpec(s, lambda t: (0,) * len(s))
    wf = pl.pallas_call(
        _filter_body,
        grid=(T,),
        in_specs=[
            pl.BlockSpec((Gb, K, Fs), lambda t: (t, 0, 0)),
            pl.BlockSpec((Gb, K), lambda t: (t, 0)),
            pl.BlockSpec((Gb, K), lambda t: (t, 0)),
            full((Fs, Ff)), full((1, Ff)), full((Ff, Ff)), full((1, Ff)),
        ],
        out_specs=pl.BlockSpec((Gb, K, Ff), lambda t: (t, 0, 0)),
        out_shape=jax.ShapeDtypeStruct((B * N, K, Ff), jnp.float32),
    )(f_ij.reshape(B * N, K, Fs), r_ij.reshape(B * N, K),
      neighbor_mask.reshape(B * N, K),
      W1.astype(jnp.bfloat16), b1.reshape(1, Ff),
      W2.astype(jnp.bfloat16), b2.reshape(1, Ff))

    nb = neighbors.astype(jnp.int32).reshape(NE)
    agg = pl.kernel(
        functools.partial(_sc_body, N=N, K=K, Ff=Ff, RPW=RPW),
        mesh=plsc.VectorSubcoreMesh(core_axis_name="c", subcore_axis_name="s"),
        compiler_params=pltpu.CompilerParams(needs_layout_passes=False),
        out_type=jax.ShapeDtypeStruct((B * N * Ff,), jnp.float32),
        scratch_types=[
            pltpu.VMEM((RPW * K,), jnp.int32),
            pltpu.VMEM((2, _R * K, Ff), jnp.float32),
            pltpu.VMEM((2, _R * K * Ff), jnp.float32),
            pltpu.VMEM((RPW * Ff,), jnp.float32),
            pltpu.SemaphoreType.DMA,
            pltpu.SemaphoreType.DMA,
        ],
    )(y, wf.reshape(NE * Ff), nb)

    Rw = 2048
    out = pl.pallas_call(
        _post_body,
        grid=((B * N) // Rw,),
        in_specs=[
            pl.BlockSpec((Rw, Ff), lambda t: (t, 0)),
            full((Ff, Dout)), full((1, Dout)),
            full((Dout, Dout)), full((1, Dout)),
        ],
        out_specs=pl.BlockSpec((Rw, Dout), lambda t: (t, 0)),
        out_shape=jax.ShapeDtypeStruct((B * N, Dout), jnp.float32),
    )(agg.reshape(B * N, Ff), Wout, bout.reshape(1, Dout),
      Wd, bd.reshape(1, Dout))

    return out.reshape(B, N, Dout)


# speedup vs baseline: 1.1795x; 1.1795x over previous
import functools
import jax
import jax.numpy as jnp
from jax import lax
from jax.experimental import pallas as pl
from jax.experimental.pallas import tpu as pltpu
from jax.experimental.pallas import tpu_sc as plsc

_LN2 = 0.6931471805599453
_CUTOFF = 5.0

_NW = 32
_R = 2
_LANES = 16
_C = 4


def _ssp(v):
    return jax.nn.softplus(v) - _LN2


def _proj_body(x_ref, Win_ref, y_ref):
    y_ref[...] = jnp.dot(x_ref[...], Win_ref[...],
                         preferred_element_type=jnp.float32)


def _filter_body(f_ref, r_ref, mask_ref, W1_ref, b1_ref, W2_ref, b2_ref,
                 wf_ref):
    f = f_ref[...].astype(jnp.bfloat16)
    h = _ssp(jnp.dot(f, W1_ref[...], preferred_element_type=jnp.float32)
             + b1_ref[...])
    wf = jnp.dot(h.astype(jnp.bfloat16), W2_ref[...],
                 preferred_element_type=jnp.float32) + b2_ref[...]
    m = jnp.where(r_ref[0, 0] <= _CUTOFF, 1.0, 0.0) * mask_ref[0, 0]
    wf_ref[...] = wf * m[:, None]


def _sc_body(y_hbm, wf_hbm, nb_hbm, out_hbm,
             idx_v, yg_v, wf_v, out_v, sem0, sem1, *, N, K, Ff, RPW, CB):
    wid = lax.axis_index("c") * 16 + lax.axis_index("s")
    base_row = CB + wid * RPW
    local_edge = wid * RPW * K
    RK = _R * K
    G = RPW // _R
    NC = Ff // _LANES

    pltpu.sync_copy(nb_hbm.at[pl.ds(base_row * K, RPW * K)], idx_v)
    bvec = jnp.full((_LANES,), (base_row // N) * N, jnp.int32)

    @pl.loop(0, (RPW * K) // _LANES)
    def _(j):
        sl = pl.ds(j * _LANES, _LANES)
        idx_v[sl] = idx_v[sl] + bvec

    sems = (sem0, sem1)

    def fire(g, slot):
        pltpu.async_copy(wf_hbm.at[pl.ds((local_edge + g * RK) * Ff, RK * Ff)],
                         wf_v.at[slot], sems[slot])
        pltpu.async_copy(y_hbm.at[idx_v.at[pl.ds(g * RK, RK)]],
                         yg_v.at[slot], sems[slot])

    def drain(slot):
        pltpu.make_async_copy(wf_hbm.at[pl.ds(0, RK * Ff)], wf_v.at[slot],
                              sems[slot]).wait()
        pltpu.make_async_copy(y_hbm.at[pl.ds(0, RK)], yg_v.at[slot],
                              sems[slot]).wait()

    def compute(g, slot):
        for r in range(_R):
            def kstep(k2, accs):
                out = list(accs)
                for dk in range(2):
                    e = r * K + k2 * 2 + dk
                    for c in range(NC):
                        w = wf_v[slot, pl.ds(e * Ff + c * _LANES, _LANES)]
                        yv = yg_v[slot, e, pl.ds(c * _LANES, _LANES)]
                        out[c] = out[c] + yv * w
                return tuple(out)
            accs = lax.fori_loop(
                0, K // 2, kstep,
                tuple(jnp.zeros((_LANES,), jnp.float32) for _ in range(NC)))
            row = g * _R + r
            for c in range(NC):
                out_v[pl.ds(row * Ff + c * _LANES, _LANES)] = accs[c]

    fire(0, 0)

    @pl.loop(0, G, step=2)
    def _(g0):
        for b in range(2):
            g = g0 + b

            @pl.when(g + 1 < G)
            def _():
                fire(g + 1, 1 - b)

            drain(b)
            compute(g, b)

    pltpu.sync_copy(out_v, out_hbm.at[pl.ds(wid * RPW * Ff, RPW * Ff)])


def _post_body(a_ref, Wout_ref, bout_ref, Wd_ref, bd_ref, o_ref):
    z = _ssp(jnp.dot(a_ref[...], Wout_ref[...],
                     preferred_element_type=jnp.float32) + bout_ref[...])
    o_ref[...] = jnp.dot(z, Wd_ref[...],
                         preferred_element_type=jnp.float32) + bd_ref[...]


@jax.jit
def kernel(x, r_ij, neighbors, neighbor_mask, f_ij,
           W1, b1, W2, b2, Win, Wout, bout, Wd, bd):
    B, N, K = neighbors.shape
    Din = x.shape[-1]
    Fs = f_ij.shape[-1]
    Ff = W2.shape[-1]
    Dout = Wd.shape[-1]
    NE = B * N * K
    AC = (B * N) // _C
    RPW = AC // _NW

    y = pl.pallas_call(
        _proj_body,
        in_specs=[pl.BlockSpec((B * N, Din), lambda: (0, 0)),
                  pl.BlockSpec((Din, Ff), lambda: (0, 0))],
        out_specs=pl.BlockSpec((B * N, Ff), lambda: (0, 0)),
        out_shape=jax.ShapeDtypeStruct((B * N, Ff), jnp.float32),
    )(x.reshape(B * N, Din), Win)

    Ew = 4096
    Tc = (AC * K) // Ew
    T = NE // Ew
    full = lambda s: pl.BlockSpec(s, lambda t: (0,) * len(s))
    f2 = f_ij.reshape(NE, Fs)
    r3 = r_ij.reshape(T, 1, Ew)
    m3 = neighbor_mask.reshape(T, 1, Ew)
    W1b = W1.astype(jnp.bfloat16)
    W2b = W2.astype(jnp.bfloat16)
    b1r = b1.reshape(1, Ff)
    b2r = b2.reshape(1, Ff)
    nb = neighbors.astype(jnp.int32).reshape(NE)

    outs = []
    for c in range(_C):
        eoff = c * Tc
        wf = pl.pallas_call(
            _filter_body,
            grid=(Tc,),
            in_specs=[
                pl.BlockSpec((Ew, Fs), lambda t, eo=eoff: (eo + t, 0)),
                pl.BlockSpec((1, 1, Ew), lambda t, eo=eoff: (eo + t, 0, 0)),
                pl.BlockSpec((1, 1, Ew), lambda t, eo=eoff: (eo + t, 0, 0)),
                full((Fs, Ff)), full((1, Ff)), full((Ff, Ff)), full((1, Ff)),
            ],
            out_specs=pl.BlockSpec((Ew, Ff), lambda t: (t, 0)),
            out_shape=jax.ShapeDtypeStruct((AC * K, Ff), jnp.float32),
        )(f2, r3, m3, W1b, b1r, W2b, b2r)

        agg = pl.kernel(
            functools.partial(_sc_body, N=N, K=K, Ff=Ff, RPW=RPW, CB=c * AC),
            mesh=plsc.VectorSubcoreMesh(core_axis_name="c",
                                        subcore_axis_name="s"),
            compiler_params=pltpu.CompilerParams(needs_layout_passes=False),
            out_type=jax.ShapeDtypeStruct((AC * Ff,), jnp.float32),
            scratch_types=[
                pltpu.VMEM((RPW * K,), jnp.int32),
                pltpu.VMEM((2, _R * K, Ff), jnp.float32),
                pltpu.VMEM((2, _R * K * Ff), jnp.float32),
                pltpu.VMEM((RPW * Ff,), jnp.float32),
                pltpu.SemaphoreType.DMA,
                pltpu.SemaphoreType.DMA,
            ],
        )(y, wf.reshape(AC * K * Ff), nb)

        out_c = pl.pallas_call(
            _post_body,
            in_specs=[
                pl.BlockSpec((AC, Ff), lambda: (0, 0)),
                pl.BlockSpec((Ff, Dout), lambda: (0, 0)),
                pl.BlockSpec((1, Dout), lambda: (0, 0)),
                pl.BlockSpec((Dout, Dout), lambda: (0, 0)),
                pl.BlockSpec((1, Dout), lambda: (0, 0)),
            ],
            out_specs=pl.BlockSpec((AC, Dout), lambda: (0, 0)),
            out_shape=jax.ShapeDtypeStruct((AC, Dout), jnp.float32),
        )(agg.reshape(AC, Ff), Wout, bout.reshape(1, Dout),
          Wd, bd.reshape(1, Dout))
        outs.append(out_c)

    return jnp.concatenate(outs, axis=0).reshape(B, N, Dout)
